# baseline (speedup 1.0000x reference)
import functools

import jax
import jax.numpy as jnp
from jax.experimental import pallas as pl
from jax.experimental.pallas import tpu as pltpu

KERNEL_SZ = 7
SCALE = 0.25
SAMPLING = 2
H = W = 64
C = 256
WIN = 16
K_BLOCK = 32


def _interp_matrix(starts, bins, size, shift=None):
    kb = starts.shape[0]
    s = jax.lax.broadcasted_iota(jnp.int32, (1, KERNEL_SZ * SAMPLING), 1).astype(jnp.float32)
    p = jnp.floor(s / 2.0)
    i = s - 2.0 * p
    offs = p + (i + 0.5) / SAMPLING
    y = starts + offs * bins
    limit = float(W if shift is not None else H)
    valid = ((y >= -1.0) & (y <= limit)).astype(jnp.float32)
    yc = jnp.clip(y, 0.0, limit - 1.0)
    y0 = jnp.floor(yc)
    y1 = jnp.minimum(y0 + 1.0, limit - 1.0)
    ly = yc - y0
    hy = 1.0 - ly
    if shift is not None:
        y0 = y0 - shift
        y1 = y1 - shift
    hh = jax.lax.broadcasted_iota(
        jnp.int32, (kb, KERNEL_SZ * SAMPLING, size), 2
    ).astype(jnp.float32)
    a = (hh == y0[:, :, None]).astype(jnp.float32) * (hy * valid)[:, :, None]
    a = a + (hh == y1[:, :, None]).astype(jnp.float32) * (ly * valid)[:, :, None]
    a = a.reshape(kb, KERNEL_SZ, SAMPLING, size).sum(axis=2) * (1.0 / SAMPLING)
    return a


def _roi_kernel(w0_ref, rois_ref, f_ref, out_ref, t_ref):
    blk = pl.program_id(0)
    r = rois_ref[...]
    sw = r[:, 1:2] * SCALE - 0.5
    sh = r[:, 2:3] * SCALE - 0.5
    ew = r[:, 3:4] * SCALE - 0.5
    eh = r[:, 4:5] * SCALE - 0.5
    bin_h = (eh - sh) * (1.0 / KERNEL_SZ)
    bin_w = (ew - sw) * (1.0 / KERNEL_SZ)

    w0f = jnp.clip(jnp.floor(sw), 0.0, float(W - WIN))

    A = _interp_matrix(sh, bin_h, H)
    Bw = _interp_matrix(sw, bin_w, WIN, shift=w0f).astype(jnp.bfloat16)

    a2 = A.reshape(K_BLOCK * KERNEL_SZ, H).astype(jnp.bfloat16)
    T = jnp.dot(a2, f_ref[...], preferred_element_type=jnp.float32)
    t_ref[...] = T.astype(jnp.bfloat16)

    pq = KERNEL_SZ * KERNEL_SZ
    for k in range(K_BLOCK):
        w0k = w0_ref[blk * K_BLOCK + k]
        tk = t_ref[k * KERNEL_SZ:(k + 1) * KERNEL_SZ, pl.ds(w0k * C, WIN * C)]
        t2 = (
            tk.reshape(KERNEL_SZ, WIN, C)
            .transpose(1, 0, 2)
            .reshape(WIN, KERNEL_SZ * C)
        )
        ok = jnp.dot(Bw[k], t2, preferred_element_type=jnp.float32)
        out_ref[k * pq:(k + 1) * pq, :] = ok.reshape(pq, C)


@jax.jit
def kernel(feats, rois):
    K = rois.shape[0]
    ft = jnp.transpose(feats[0], (1, 2, 0)).reshape(H, W * C).astype(jnp.bfloat16)
    w0 = jnp.clip(
        jnp.floor(rois[:, 1] * SCALE - 0.5), 0.0, float(W - WIN)
    ).astype(jnp.int32)
    grid = K // K_BLOCK
    pq = KERNEL_SZ * KERNEL_SZ
    out = pl.pallas_call(
        _roi_kernel,
        grid_spec=pltpu.PrefetchScalarGridSpec(
            num_scalar_prefetch=1,
            grid=(grid,),
            in_specs=[
                pl.BlockSpec((K_BLOCK, 5), lambda i, w0_ref: (i, 0)),
                pl.BlockSpec((H, W * C), lambda i, w0_ref: (0, 0)),
            ],
            out_specs=pl.BlockSpec((K_BLOCK * pq, C), lambda i, w0_ref: (i, 0)),
            scratch_shapes=[pltpu.VMEM((K_BLOCK * KERNEL_SZ, W * C), jnp.bfloat16)],
        ),
        out_shape=jax.ShapeDtypeStruct((K * pq, C), jnp.float32),
        compiler_params=pltpu.CompilerParams(
            dimension_semantics=("parallel",),
        ),
    )(w0, rois, ft)
    out = out.reshape(K, KERNEL_SZ, KERNEL_SZ, C).transpose(0, 3, 2, 1)
    return out


# speedup vs baseline: 1.2584x; 1.2584x over previous
import functools

import jax
import jax.numpy as jnp
from jax.experimental import pallas as pl
from jax.experimental.pallas import tpu as pltpu

KERNEL_SZ = 7
SCALE = 0.25
SAMPLING = 2
H = W = 64
C = 256
WIN = 16
K_BLOCK = 32


def _interp_matrix(starts, bins, size, shift=None):
    kb = starts.shape[0]
    s = jax.lax.broadcasted_iota(jnp.int32, (1, KERNEL_SZ * SAMPLING), 1).astype(jnp.float32)
    p = jnp.floor(s / 2.0)
    i = s - 2.0 * p
    offs = p + (i + 0.5) / SAMPLING
    y = starts + offs * bins
    limit = float(W if shift is not None else H)
    valid = ((y >= -1.0) & (y <= limit)).astype(jnp.float32)
    yc = jnp.clip(y, 0.0, limit - 1.0)
    y0 = jnp.floor(yc)
    y1 = jnp.minimum(y0 + 1.0, limit - 1.0)
    ly = yc - y0
    hy = 1.0 - ly
    if shift is not None:
        y0 = y0 - shift
        y1 = y1 - shift
    hh = jax.lax.broadcasted_iota(
        jnp.int32, (kb, KERNEL_SZ * SAMPLING, size), 2
    ).astype(jnp.float32)
    a = (hh == y0[:, :, None]).astype(jnp.float32) * (hy * valid)[:, :, None]
    a = a + (hh == y1[:, :, None]).astype(jnp.float32) * (ly * valid)[:, :, None]
    a = a.reshape(kb, KERNEL_SZ, SAMPLING, size).sum(axis=2) * (1.0 / SAMPLING)
    return a


def _roi_kernel(w0_ref, rois_ref, f_ref, out_ref, t_ref):
    blk = pl.program_id(0)
    r = rois_ref[...]
    sw = r[:, 1:2] * SCALE - 0.5
    sh = r[:, 2:3] * SCALE - 0.5
    ew = r[:, 3:4] * SCALE - 0.5
    eh = r[:, 4:5] * SCALE - 0.5
    bin_h = (eh - sh) * (1.0 / KERNEL_SZ)
    bin_w = (ew - sw) * (1.0 / KERNEL_SZ)

    w0f = jnp.clip(jnp.floor(sw), 0.0, float(W - WIN))

    A = _interp_matrix(sh, bin_h, H)
    Bw = _interp_matrix(sw, bin_w, WIN, shift=w0f).astype(jnp.bfloat16)

    a2 = A.reshape(K_BLOCK * KERNEL_SZ, H).astype(jnp.bfloat16)
    T = jnp.dot(a2, f_ref[...], preferred_element_type=jnp.float32)
    t_ref[...] = T.astype(jnp.bfloat16)

    for k in range(K_BLOCK):
        w0k = w0_ref[blk * K_BLOCK + k]
        tk = t_ref[k * KERNEL_SZ:(k + 1) * KERNEL_SZ, pl.ds(w0k * C, WIN * C)]
        t2 = (
            tk.reshape(KERNEL_SZ, WIN, C)
            .transpose(1, 0, 2)
            .reshape(WIN, KERNEL_SZ * C)
        )
        ok = jnp.dot(Bw[k], t2, preferred_element_type=jnp.float32)
        out_ref[k * KERNEL_SZ:(k + 1) * KERNEL_SZ] = ok.reshape(
            KERNEL_SZ, KERNEL_SZ, C
        )


@jax.jit
def kernel(feats, rois):
    K = rois.shape[0]
    ft = jnp.transpose(feats[0], (1, 2, 0)).reshape(H, W * C).astype(jnp.bfloat16)
    w0 = jnp.clip(
        jnp.floor(rois[:, 1] * SCALE - 0.5), 0.0, float(W - WIN)
    ).astype(jnp.int32)
    grid = K // K_BLOCK
    pq = KERNEL_SZ * KERNEL_SZ
    out = pl.pallas_call(
        _roi_kernel,
        grid_spec=pltpu.PrefetchScalarGridSpec(
            num_scalar_prefetch=1,
            grid=(grid,),
            in_specs=[
                pl.BlockSpec((K_BLOCK, 5), lambda i, w0_ref: (i, 0)),
                pl.BlockSpec((H, W * C), lambda i, w0_ref: (0, 0)),
            ],
            out_specs=pl.BlockSpec(
                (K_BLOCK * KERNEL_SZ, KERNEL_SZ, C), lambda i, w0_ref: (i, 0, 0)
            ),
            scratch_shapes=[pltpu.VMEM((K_BLOCK * KERNEL_SZ, W * C), jnp.bfloat16)],
        ),
        out_shape=jax.ShapeDtypeStruct((K * KERNEL_SZ, KERNEL_SZ, C), jnp.float32),
        compiler_params=pltpu.CompilerParams(
            dimension_semantics=("parallel",),
        ),
    )(w0, rois, ft)
    out = out.reshape(K, KERNEL_SZ, KERNEL_SZ, C).transpose(0, 3, 2, 1)
    return out


# speedup vs baseline: 1.3040x; 1.0363x over previous
import functools

import jax
import jax.numpy as jnp
from jax.experimental import pallas as pl
from jax.experimental.pallas import tpu as pltpu

KERNEL_SZ = 7
SCALE = 0.25
SAMPLING = 2
H = W = 64
C = 256
WIN = 16
K_BLOCK = 64


def _interp_matrix(starts, bins, size, shift=None):
    kb = starts.shape[0]
    s = jax.lax.broadcasted_iota(jnp.int32, (1, KERNEL_SZ * SAMPLING), 1).astype(jnp.float32)
    p = jnp.floor(s / 2.0)
    i = s - 2.0 * p
    offs = p + (i + 0.5) / SAMPLING
    y = starts + offs * bins
    limit = float(W if shift is not None else H)
    valid = ((y >= -1.0) & (y <= limit)).astype(jnp.float32)
    yc = jnp.clip(y, 0.0, limit - 1.0)
    y0 = jnp.floor(yc)
    y1 = jnp.minimum(y0 + 1.0, limit - 1.0)
    ly = yc - y0
    hy = 1.0 - ly
    if shift is not None:
        y0 = y0 - shift
        y1 = y1 - shift
    hh = jax.lax.broadcasted_iota(
        jnp.int32, (kb, KERNEL_SZ * SAMPLING, size), 2
    ).astype(jnp.float32)
    a = (hh == y0[:, :, None]).astype(jnp.float32) * (hy * valid)[:, :, None]
    a = a + (hh == y1[:, :, None]).astype(jnp.float32) * (ly * valid)[:, :, None]
    a = a.reshape(kb, KERNEL_SZ, SAMPLING, size).sum(axis=2) * (1.0 / SAMPLING)
    return a


def _roi_kernel(w0_ref, rois_ref, f_ref, out_ref, t_ref):
    blk = pl.program_id(0)
    r = rois_ref[...]
    sw = r[:, 1:2] * SCALE - 0.5
    sh = r[:, 2:3] * SCALE - 0.5
    ew = r[:, 3:4] * SCALE - 0.5
    eh = r[:, 4:5] * SCALE - 0.5
    bin_h = (eh - sh) * (1.0 / KERNEL_SZ)
    bin_w = (ew - sw) * (1.0 / KERNEL_SZ)

    w0f = jnp.clip(jnp.floor(sw), 0.0, float(W - WIN))

    A = _interp_matrix(sh, bin_h, H)
    Bw = _interp_matrix(sw, bin_w, WIN, shift=w0f).astype(jnp.bfloat16)

    a2 = A.reshape(K_BLOCK * KERNEL_SZ, H).astype(jnp.bfloat16)
    T = jnp.dot(a2, f_ref[...], preferred_element_type=jnp.float32)
    t_ref[...] = T.astype(jnp.bfloat16)

    for k in range(K_BLOCK):
        w0k = w0_ref[blk * K_BLOCK + k]
        tk = t_ref[k * KERNEL_SZ:(k + 1) * KERNEL_SZ, pl.ds(w0k * C, WIN * C)]
        t2 = (
            tk.reshape(KERNEL_SZ, WIN, C)
            .transpose(1, 0, 2)
            .reshape(WIN, KERNEL_SZ * C)
        )
        ok = jnp.dot(Bw[k], t2, preferred_element_type=jnp.float32)
        out_ref[k * KERNEL_SZ:(k + 1) * KERNEL_SZ] = ok.reshape(
            KERNEL_SZ, KERNEL_SZ, C
        )


@jax.jit
def kernel(feats, rois):
    K = rois.shape[0]
    ft = jnp.transpose(feats[0], (1, 2, 0)).reshape(H, W * C).astype(jnp.bfloat16)
    w0 = jnp.clip(
        jnp.floor(rois[:, 1] * SCALE - 0.5), 0.0, float(W - WIN)
    ).astype(jnp.int32)
    grid = K // K_BLOCK
    pq = KERNEL_SZ * KERNEL_SZ
    out = pl.pallas_call(
        _roi_kernel,
        grid_spec=pltpu.PrefetchScalarGridSpec(
            num_scalar_prefetch=1,
            grid=(grid,),
            in_specs=[
                pl.BlockSpec((K_BLOCK, 5), lambda i, w0_ref: (i, 0)),
                pl.BlockSpec((H, W * C), lambda i, w0_ref: (0, 0)),
            ],
            out_specs=pl.BlockSpec(
                (K_BLOCK * KERNEL_SZ, KERNEL_SZ, C), lambda i, w0_ref: (i, 0, 0)
            ),
            scratch_shapes=[pltpu.VMEM((K_BLOCK * KERNEL_SZ, W * C), jnp.bfloat16)],
        ),
        out_shape=jax.ShapeDtypeStruct((K * KERNEL_SZ, KERNEL_SZ, C), jnp.float32),
        compiler_params=pltpu.CompilerParams(
            dimension_semantics=("parallel",),
        ),
    )(w0, rois, ft)
    out = out.reshape(K, KERNEL_SZ, KERNEL_SZ, C).transpose(0, 3, 2, 1)
    return out


# speedup vs baseline: 1.3186x; 1.0111x over previous
import functools

import jax
import jax.numpy as jnp
from jax.experimental import pallas as pl
from jax.experimental.pallas import tpu as pltpu

KERNEL_SZ = 7
SCALE = 0.25
SAMPLING = 2
H = W = 64
C = 256
WIN = 16
K_BLOCK = 128


def _interp_matrix(starts, bins, size, shift=None):
    kb = starts.shape[0]
    s = jax.lax.broadcasted_iota(jnp.int32, (1, KERNEL_SZ * SAMPLING), 1).astype(jnp.float32)
    p = jnp.floor(s / 2.0)
    i = s - 2.0 * p
    offs = p + (i + 0.5) / SAMPLING
    y = starts + offs * bins
    limit = float(W if shift is not None else H)
    valid = ((y >= -1.0) & (y <= limit)).astype(jnp.float32)
    yc = jnp.clip(y, 0.0, limit - 1.0)
    y0 = jnp.floor(yc)
    y1 = jnp.minimum(y0 + 1.0, limit - 1.0)
    ly = yc - y0
    hy = 1.0 - ly
    if shift is not None:
        y0 = y0 - shift
        y1 = y1 - shift
    hh = jax.lax.broadcasted_iota(
        jnp.int32, (kb, KERNEL_SZ * SAMPLING, size), 2
    ).astype(jnp.float32)
    a = (hh == y0[:, :, None]).astype(jnp.float32) * (hy * valid)[:, :, None]
    a = a + (hh == y1[:, :, None]).astype(jnp.float32) * (ly * valid)[:, :, None]
    a = a.reshape(kb, KERNEL_SZ, SAMPLING, size).sum(axis=2) * (1.0 / SAMPLING)
    return a


def _roi_kernel(w0_ref, rois_ref, f_ref, out_ref, t_ref):
    blk = pl.program_id(0)
    r = rois_ref[...]
    sw = r[:, 1:2] * SCALE - 0.5
    sh = r[:, 2:3] * SCALE - 0.5
    ew = r[:, 3:4] * SCALE - 0.5
    eh = r[:, 4:5] * SCALE - 0.5
    bin_h = (eh - sh) * (1.0 / KERNEL_SZ)
    bin_w = (ew - sw) * (1.0 / KERNEL_SZ)

    w0f = jnp.clip(jnp.floor(sw), 0.0, float(W - WIN))

    A = _interp_matrix(sh, bin_h, H)
    Bw = _interp_matrix(sw, bin_w, WIN, shift=w0f).astype(jnp.bfloat16)

    a2 = A.reshape(K_BLOCK * KERNEL_SZ, H).astype(jnp.bfloat16)
    T = jnp.dot(a2, f_ref[...], preferred_element_type=jnp.float32)
    t_ref[...] = T.astype(jnp.bfloat16)

    for k in range(K_BLOCK):
        w0k = w0_ref[blk * K_BLOCK + k]
        tk = t_ref[k * KERNEL_SZ:(k + 1) * KERNEL_SZ, pl.ds(w0k * C, WIN * C)]
        t2 = (
            tk.reshape(KERNEL_SZ, WIN, C)
            .transpose(1, 0, 2)
            .reshape(WIN, KERNEL_SZ * C)
        )
        ok = jnp.dot(Bw[k], t2, preferred_element_type=jnp.float32)
        out_ref[k * KERNEL_SZ:(k + 1) * KERNEL_SZ] = ok.reshape(
            KERNEL_SZ, KERNEL_SZ, C
        )


@jax.jit
def kernel(feats, rois):
    K = rois.shape[0]
    ft = jnp.transpose(feats[0], (1, 2, 0)).reshape(H, W * C).astype(jnp.bfloat16)
    w0 = jnp.clip(
        jnp.floor(rois[:, 1] * SCALE - 0.5), 0.0, float(W - WIN)
    ).astype(jnp.int32)
    grid = K // K_BLOCK
    pq = KERNEL_SZ * KERNEL_SZ
    out = pl.pallas_call(
        _roi_kernel,
        grid_spec=pltpu.PrefetchScalarGridSpec(
            num_scalar_prefetch=1,
            grid=(grid,),
            in_specs=[
                pl.BlockSpec((K_BLOCK, 5), lambda i, w0_ref: (i, 0)),
                pl.BlockSpec((H, W * C), lambda i, w0_ref: (0, 0)),
            ],
            out_specs=pl.BlockSpec(
                (K_BLOCK * KERNEL_SZ, KERNEL_SZ, C), lambda i, w0_ref: (i, 0, 0)
            ),
            scratch_shapes=[pltpu.VMEM((K_BLOCK * KERNEL_SZ, W * C), jnp.bfloat16)],
        ),
        out_shape=jax.ShapeDtypeStruct((K * KERNEL_SZ, KERNEL_SZ, C), jnp.float32),
        compiler_params=pltpu.CompilerParams(
            dimension_semantics=("parallel",),
        ),
    )(w0, rois, ft)
    out = out.reshape(K, KERNEL_SZ, KERNEL_SZ, C).transpose(0, 3, 2, 1)
    return out
